# baseline (speedup 1.0000x reference)
import functools

import jax
import jax.numpy as jnp
from jax import lax
from jax.experimental import pallas as pl
from jax.experimental.pallas import tpu as pltpu
from jax.experimental.pallas import tpu_sc as plsc

B = 16384
L = 200
V = 7
D = 4
NW = 32
TCB = 8192
SCB = B - TCB
RPW = SCB // NW
CB = 128
NCHUNK = RPW // CB
NL4 = (V ** 4 + 15) // 16
PAIR_STEP = 4
TCBLK = 512


def _sc_body(xt_hbm, params_hbm, out_hbm,
             xv, lut1, lut2, lut4, outv, pv, sems):
    nc = 2
    wid = lax.axis_index("s") * nc + lax.axis_index("c")
    base = TCB + wid * RPW

    pltpu.sync_copy(params_hbm, pv)

    lane = lax.iota(jnp.int32, 16)

    k_ix = jnp.minimum(lane, V - 1) * D
    v = jnp.zeros((16,), jnp.float32)
    for j in range(D):
        tcol = plsc.load_gather(pv, [k_ix + j])
        wj = plsc.load_gather(pv, [jnp.full((16,), V * D + j, jnp.int32)])
        v = v + tcol * wj
    lut1[...] = v * (1.0 / L)
    bias = plsc.load_gather(pv, [jnp.full((16,), V * D + D, jnp.int32)])

    for i in range(4):
        ix = lane + i * 16
        lut2[pl.ds(i * 16, 16)] = (plsc.load_gather(lut1, [ix // V]) +
                                   plsc.load_gather(lut1, [ix % V]))

    def l4_body(i, carry):
        ix = lane + i * 16
        lut4[pl.ds(i * 16, 16)] = (
            plsc.load_gather(lut2, [ix // (V * V)]) +
            plsc.load_gather(lut2, [ix % (V * V)]))
        return carry

    lax.fori_loop(0, NL4, l4_body, 0)

    def make_chunk_compute(buf, chunk):
        def group_body(g, carry):
            b0 = g * 16

            def pack(l):
                a0 = buf[l, pl.ds(b0, 16)]
                a1 = buf[l + 1, pl.ds(b0, 16)]
                a2 = buf[l + 2, pl.ds(b0, 16)]
                a3 = buf[l + 3, pl.ds(b0, 16)]
                ix = ((a0 * V + a1) * V + a2) * V + a3
                return plsc.load_gather(lut4, [ix])

            def col_body(i, accs):
                acc0, acc1 = accs
                l = i * (2 * PAIR_STEP)
                return (acc0 + pack(l), acc1 + pack(l + PAIR_STEP))

            zacc = jnp.zeros((16,), jnp.float32)
            acc0, acc1 = lax.fori_loop(0, L // (2 * PAIR_STEP), col_body,
                                       (zacc, zacc), unroll=5)
            z = acc0 + acc1 + bias
            outv[pl.ds(chunk * CB + b0, 16)] = 1.0 / (1.0 + jnp.exp(-z))
            return carry
        return group_body

    copies = [None] * NCHUNK
    copies[0] = pltpu.async_copy(
        xt_hbm.at[:, pl.ds(base, CB)], xv.at[0], sems.at[0])
    for c in range(NCHUNK):
        if c + 1 < NCHUNK:
            copies[c + 1] = pltpu.async_copy(
                xt_hbm.at[:, pl.ds(base + (c + 1) * CB, CB)],
                xv.at[(c + 1) % 2], sems.at[(c + 1) % 2])
        copies[c].wait()
        lax.fori_loop(0, CB // 16, make_chunk_compute(xv.at[c % 2], c), 0)

    pltpu.sync_copy(outv, out_hbm.at[pl.ds(wid * RPW, RPW)])


@functools.partial(
    pl.kernel,
    out_type=jax.ShapeDtypeStruct((SCB,), jnp.float32),
    mesh=plsc.VectorSubcoreMesh(core_axis_name="c", subcore_axis_name="s"),
    scratch_types=[
        pltpu.VMEM((2, L, CB), jnp.int32),
        pltpu.VMEM((16,), jnp.float32),
        pltpu.VMEM((64,), jnp.float32),
        pltpu.VMEM((NL4 * 16,), jnp.float32),
        pltpu.VMEM((RPW,), jnp.float32),
        pltpu.VMEM((48,), jnp.float32),
        pltpu.SemaphoreType.DMA((2,)),
    ],
    compiler_params=pltpu.CompilerParams(
        needs_layout_passes=False, disable_bounds_checks=True),
)
def _shadow_sc(*args):
    _sc_body(*args)


def _tc_body(xt_ref, tbl_ref, w_ref, b_ref, out_ref):
    i = pl.program_id(0)
    tbl = tbl_ref[...]
    w = w_ref[...]
    vk = jnp.sum(tbl * w[:, 0][None, :], axis=1) * (1.0 / L)
    x_blk = xt_ref[...]
    b0 = (x_blk & 1) != 0
    b1 = (x_blk & 2) != 0
    b2 = (x_blk & 4) != 0
    t01 = jnp.where(b0, vk[1], vk[0])
    t23 = jnp.where(b0, vk[3], vk[2])
    t45 = jnp.where(b0, vk[5], vk[4])
    lo = jnp.where(b1, t23, t01)
    hi = jnp.where(b1, vk[6], t45)
    vals = jnp.where(b2, hi, lo)
    s = jnp.sum(vals, axis=0, keepdims=True)

    @pl.when(i == 0)
    def _():
        out_ref[...] = s

    @pl.when(i > 0)
    def _():
        out_ref[...] = out_ref[...] + s

    @pl.when(i == L // 8 - 1)
    def _():
        z = out_ref[...] + b_ref[0]
        out_ref[...] = 1.0 / (1.0 + jnp.exp(-z))


_shadow_tc = pl.pallas_call(
    _tc_body,
    grid=(L // 8,),
    in_specs=[
        pl.BlockSpec((8, TCB), lambda i: (i, 0)),
        pl.BlockSpec((V, D), lambda i: (0, 0)),
        pl.BlockSpec((D, 1), lambda i: (0, 0)),
        pl.BlockSpec((1,), lambda i: (0,)),
    ],
    out_specs=pl.BlockSpec((1, TCB), lambda i: (0, 0)),
    out_shape=jax.ShapeDtypeStruct((1, TCB), jnp.float32),
)


def kernel(x, table, W, b):
    xt = x.T
    params = jnp.concatenate([table.reshape(-1), W.reshape(-1), b])
    params = jnp.pad(params, (0, 48 - params.shape[0]))
    sc_out = _shadow_sc(xt, params)
    tc_out = _shadow_tc(xt, table, W, b)
    return jnp.concatenate([tc_out.reshape(TCB), sc_out]).reshape(B, 1)


# speedup vs baseline: 1.2801x; 1.2801x over previous
import functools

import jax
import jax.numpy as jnp
from jax import lax
from jax.experimental import pallas as pl
from jax.experimental.pallas import tpu as pltpu
from jax.experimental.pallas import tpu_sc as plsc

B = 16384
L = 200
V = 7
D = 4
NW = 32
RPW = B // NW
CB = 128
NCHUNK = RPW // CB
NL4 = (V ** 4 + 15) // 16
PAIR_STEP = 4


def _sc_body(xt_hbm, params_hbm, out_hbm,
             xv, lut1, lut2, lut4, outv, pv, sems):
    nc = 2
    wid = lax.axis_index("s") * nc + lax.axis_index("c")
    base = wid * RPW

    pltpu.sync_copy(params_hbm, pv)

    lane = lax.iota(jnp.int32, 16)

    k_ix = jnp.minimum(lane, V - 1) * D
    v = jnp.zeros((16,), jnp.float32)
    for j in range(D):
        tcol = plsc.load_gather(pv, [k_ix + j])
        wj = plsc.load_gather(pv, [jnp.full((16,), V * D + j, jnp.int32)])
        v = v + tcol * wj
    lut1[...] = v * (1.0 / L)
    bias = plsc.load_gather(pv, [jnp.full((16,), V * D + D, jnp.int32)])

    for i in range(4):
        ix = lane + i * 16
        lut2[pl.ds(i * 16, 16)] = (plsc.load_gather(lut1, [ix // V]) +
                                   plsc.load_gather(lut1, [ix % V]))

    def l4_body(i, carry):
        ix = lane + i * 16
        lut4[pl.ds(i * 16, 16)] = (
            plsc.load_gather(lut2, [ix // (V * V)]) +
            plsc.load_gather(lut2, [ix % (V * V)]))
        return carry

    lax.fori_loop(0, NL4, l4_body, 0)

    def make_chunk_compute(buf, chunk):
        def group_body(g, carry):
            b0 = g * 16

            def col_body(i, acc):
                l = i * PAIR_STEP
                a0 = buf[l, pl.ds(b0, 16)]
                a1 = buf[l + 1, pl.ds(b0, 16)]
                a2 = buf[l + 2, pl.ds(b0, 16)]
                a3 = buf[l + 3, pl.ds(b0, 16)]
                ix = ((a0 * V + a1) * V + a2) * V + a3
                return acc + plsc.load_gather(lut4, [ix])

            acc = lax.fori_loop(0, L // PAIR_STEP, col_body,
                                jnp.zeros((16,), jnp.float32), unroll=5)
            z = acc + bias
            outv[pl.ds(chunk * CB + b0, 16)] = 1.0 / (1.0 + jnp.exp(-z))
            return carry
        return group_body

    copies = [None] * NCHUNK
    copies[0] = pltpu.async_copy(
        xt_hbm.at[:, pl.ds(base, CB)], xv.at[0], sems.at[0])
    for c in range(NCHUNK):
        if c + 1 < NCHUNK:
            copies[c + 1] = pltpu.async_copy(
                xt_hbm.at[:, pl.ds(base + (c + 1) * CB, CB)],
                xv.at[(c + 1) % 2], sems.at[(c + 1) % 2])
        copies[c].wait()
        lax.fori_loop(0, CB // 16, make_chunk_compute(xv.at[c % 2], c), 0)

    pltpu.sync_copy(outv, out_hbm.at[pl.ds(base, RPW)])


@functools.partial(
    pl.kernel,
    out_type=jax.ShapeDtypeStruct((B,), jnp.float32),
    mesh=plsc.VectorSubcoreMesh(core_axis_name="c", subcore_axis_name="s"),
    scratch_types=[
        pltpu.VMEM((2, L, CB), jnp.int32),
        pltpu.VMEM((16,), jnp.float32),
        pltpu.VMEM((64,), jnp.float32),
        pltpu.VMEM((NL4 * 16,), jnp.float32),
        pltpu.VMEM((RPW,), jnp.float32),
        pltpu.VMEM((48,), jnp.float32),
        pltpu.SemaphoreType.DMA((2,)),
    ],
    compiler_params=pltpu.CompilerParams(
        needs_layout_passes=False, disable_bounds_checks=True),
)
def _shadow_sc(*args):
    _sc_body(*args)


def kernel(x, table, W, b):
    params = jnp.concatenate([table.reshape(-1), W.reshape(-1), b])
    params = jnp.pad(params, (0, 48 - params.shape[0]))
    out = _shadow_sc(x.T, params)
    return out.reshape(B, 1)
